# baseline (speedup 1.0000x reference)
import jax
import jax.numpy as jnp
from jax.experimental import pallas as pl
from jax.experimental.pallas import tpu as pltpu

_BI = 400
_BH = 200
_RC = 2
_CC = 2048

_DN = (((1,), (1,)), ((), ()))
_DC = (((1,), (0,)), ((), ()))


def _body(adj_t_ref, adj_b_ref, feat_ref, w1_ref, w2_ref, bias_ref,
          pre_ref, out_ref, cache_ref, g16_ref, pre1_ref):
    p = pl.program_id(0)
    i = pl.program_id(1)
    N = feat_ref.shape[0]

    @pl.when(p == 0)
    def _pass1():
        @pl.when(i < _RC)
        def _():
            for half, ref in ((0, adj_t_ref), (1, adj_b_ref)):
                for c in range(0, N, _CC):
                    w = min(_CC, N - c)
                    cache_ref[pl.ds(i * _BI + half * _BH, _BH),
                              pl.ds(c, w)] = (
                        ref[:, pl.ds(c, w)].astype(jnp.bfloat16))

        for half, ref in ((0, adj_t_ref), (1, adj_b_ref)):
            r0 = i * _BI + half * _BH
            ax = jnp.dot(ref[...], feat_ref[...],
                         preferred_element_type=jnp.float32)
            g16_ref[pl.ds(r0, _BH), :] = (
                ax * feat_ref[pl.ds(r0, _BH), :]).astype(jnp.bfloat16)
            pre1_ref[pl.ds(r0, _BH), :] = jax.lax.dot_general(
                ax, w1_ref[...], _DN,
                precision=jax.lax.Precision.HIGHEST,
                preferred_element_type=jnp.float32).astype(jnp.bfloat16)

    @pl.when(p == 1)
    def _pass2():
        def finish(half, axx):
            pre = (
                pre1_ref[pl.ds(i * _BI + half * _BH, _BH), :].astype(
                    jnp.float32)
                + jax.lax.dot_general(
                    axx, w2_ref[...], _DN,
                    precision=jax.lax.Precision.HIGHEST,
                    preferred_element_type=jnp.float32)
                + bias_ref[...]
            )
            pre_ref[pl.ds(half * _BH, _BH), :] = pre
            out_ref[pl.ds(half * _BH, _BH), :] = jnp.where(
                pre > 0, pre, jnp.exp(pre) - 1.0)

        @pl.when(i < _RC)
        def _():
            for half in (0, 1):
                finish(half, jnp.dot(
                    cache_ref[pl.ds(i * _BI + half * _BH, _BH), :],
                    g16_ref[...], preferred_element_type=jnp.float32))

        @pl.when(i >= _RC)
        def _():
            for half, ref in ((0, adj_t_ref), (1, adj_b_ref)):
                finish(half, jax.lax.dot_general(
                    ref[...], g16_ref[...], _DC,
                    preferred_element_type=jnp.float32))


def kernel(features, adj, W1, b1, W2, b2):
    N, H = features.shape
    R = N // _BI
    bias = (b1 + b2).reshape(1, H)

    def top_map(p, i):
        return (2 * jnp.maximum(i, p * _RC), 0)

    def bot_map(p, i):
        return (2 * jnp.maximum(i, p * _RC) + 1, 0)

    pre, out = pl.pallas_call(
        _body,
        grid=(2, R),
        in_specs=[
            pl.BlockSpec((_BH, N), top_map),
            pl.BlockSpec((_BH, N), bot_map),
            pl.BlockSpec((N, H), lambda p, i: (0, 0)),
            pl.BlockSpec((H, H), lambda p, i: (0, 0)),
            pl.BlockSpec((H, H), lambda p, i: (0, 0)),
            pl.BlockSpec((1, H), lambda p, i: (0, 0)),
        ],
        out_specs=[
            pl.BlockSpec((_BI, H), lambda p, i: (i * p, 0)),
            pl.BlockSpec((_BI, H), lambda p, i: (i * p, 0)),
        ],
        out_shape=[
            jax.ShapeDtypeStruct((N, H), jnp.float32),
            jax.ShapeDtypeStruct((N, H), jnp.float32),
        ],
        scratch_shapes=[
            pltpu.VMEM((_RC * _BI, N), jnp.bfloat16),
            pltpu.VMEM((N, H), jnp.bfloat16),
            pltpu.VMEM((N, H), jnp.bfloat16),
        ],
    )(adj, adj, features, W1, W2, bias)
    return (pre, out)


# speedup vs baseline: 1.1052x; 1.1052x over previous
import jax
import jax.numpy as jnp
from jax.experimental import pallas as pl
from jax.experimental.pallas import tpu as pltpu

_BI = 400
_RC = 2
_CC = 2048


def _body(adj_ref, feat_ref, w1_ref, w2_ref, bias_ref,
          pre_ref, out_ref, cache_ref, g16_ref, pre1_ref):
    p = pl.program_id(0)
    i = pl.program_id(1)
    N = feat_ref.shape[0]
    dn = (((1,), (1,)), ((), ()))

    @pl.when(p == 0)
    def _pass1():
        @pl.when(i < _RC)
        def _():
            for c in range(0, N, _CC):
                w = min(_CC, N - c)
                cache_ref[pl.ds(i * _BI, _BI), pl.ds(c, w)] = (
                    adj_ref[:, pl.ds(c, w)].astype(jnp.bfloat16))

        ax = jnp.dot(adj_ref[...], feat_ref[...],
                     preferred_element_type=jnp.float32)
        g = ax * feat_ref[pl.ds(i * _BI, _BI), :]
        g16_ref[pl.ds(i * _BI, _BI), :] = g.astype(jnp.bfloat16)
        pre1_ref[pl.ds(i * _BI, _BI), :] = jax.lax.dot_general(
            ax, w1_ref[...], dn,
            precision=jax.lax.Precision.HIGHEST,
            preferred_element_type=jnp.float32).astype(jnp.bfloat16)

    @pl.when(p == 1)
    def _pass2():
        def finish(axx):
            pre = (
                pre1_ref[pl.ds(i * _BI, _BI), :].astype(jnp.float32)
                + jax.lax.dot_general(
                    axx, w2_ref[...], dn,
                    precision=jax.lax.Precision.HIGHEST,
                    preferred_element_type=jnp.float32)
                + bias_ref[...]
            )
            pre_ref[...] = pre
            out_ref[...] = jnp.where(pre > 0, pre, jnp.exp(pre) - 1.0)

        @pl.when(i < _RC)
        def _():
            finish(jnp.dot(cache_ref[pl.ds(i * _BI, _BI), :], g16_ref[...],
                           preferred_element_type=jnp.float32))

        @pl.when(i >= _RC)
        def _():
            finish(jax.lax.dot_general(
                adj_ref[...], g16_ref[...], (((1,), (0,)), ((), ())),
                preferred_element_type=jnp.float32))


def kernel(features, adj, W1, b1, W2, b2):
    N, H = features.shape
    R = N // _BI
    bias = (b1 + b2).reshape(1, H)

    pre, out = pl.pallas_call(
        _body,
        grid=(2, R),
        in_specs=[
            pl.BlockSpec((_BI, N), lambda p, i: (jnp.maximum(i, p * _RC), 0)),
            pl.BlockSpec((N, H), lambda p, i: (0, 0)),
            pl.BlockSpec((H, H), lambda p, i: (0, 0)),
            pl.BlockSpec((H, H), lambda p, i: (0, 0)),
            pl.BlockSpec((1, H), lambda p, i: (0, 0)),
        ],
        out_specs=[
            pl.BlockSpec((_BI, H), lambda p, i: (i * p, 0)),
            pl.BlockSpec((_BI, H), lambda p, i: (i * p, 0)),
        ],
        out_shape=[
            jax.ShapeDtypeStruct((N, H), jnp.float32),
            jax.ShapeDtypeStruct((N, H), jnp.float32),
        ],
        scratch_shapes=[
            pltpu.VMEM((_RC * _BI, N), jnp.bfloat16),
            pltpu.VMEM((N, H), jnp.bfloat16),
            pltpu.VMEM((N, H), jnp.bfloat16),
        ],
    )(adj, features, W1, W2, bias)
    return (pre, out)


# speedup vs baseline: 1.1332x; 1.0253x over previous
import jax
import jax.numpy as jnp
from jax.experimental import pallas as pl
from jax.experimental.pallas import tpu as pltpu

_BI = 400
_RC = 2
_CC = 2048
_PC = 512


def _make_body(N, R):
    NCH = -(-N // _PC)
    WLAST = N - _PC * (NCH - 1)
    dn = (((1,), (1,)), ((), ()))
    dc = (((1,), (0,)), ((), ()))

    def body(adj_ref, feat_ref, w1_ref, w2_ref, bias_ref,
             pre_ref, out_ref, cache_ref, g16_ref, pre1_ref, accc_ref):
        p = pl.program_id(0)
        i = pl.program_id(1)

        @pl.when(p == 0)
        def _pass1():
            @pl.when(i < _RC)
            def _():
                for c in range(0, N, _CC):
                    w = min(_CC, N - c)
                    cache_ref[pl.ds(i * _BI, _BI), pl.ds(c, w)] = (
                        adj_ref[:, pl.ds(c, w)].astype(jnp.bfloat16))

            ax = jnp.dot(adj_ref[...], feat_ref[...],
                         preferred_element_type=jnp.float32)
            g = ax * feat_ref[pl.ds(i * _BI, _BI), :]
            g16_ref[pl.ds(i * _BI, _BI), :] = g.astype(jnp.bfloat16)
            pre1_ref[pl.ds(i * _BI, _BI), :] = jax.lax.dot_general(
                ax, w1_ref[...], dn,
                precision=jax.lax.Precision.HIGHEST,
                preferred_element_type=jnp.float32).astype(jnp.bfloat16)

        @pl.when(p == 1)
        def _pass2():
            def finish(row0, axx):
                pre = (
                    pre1_ref[pl.ds(row0, _BI), :].astype(jnp.float32)
                    + jax.lax.dot_general(
                        axx, w2_ref[...], dn,
                        precision=jax.lax.Precision.HIGHEST,
                        preferred_element_type=jnp.float32)
                    + bias_ref[...]
                )
                pre_ref[...] = pre
                out_ref[...] = jnp.where(pre > 0, pre, jnp.exp(pre) - 1.0)

            def cached_partial(c0, w):
                return jnp.dot(
                    cache_ref[:, pl.ds(c0, w)],
                    g16_ref[pl.ds(c0, w), :],
                    preferred_element_type=jnp.float32)

            @pl.when(i == 0)
            def _():
                accc_ref[...] = cached_partial(0, _PC)

            @pl.when((i > 0) & (i < NCH - 1))
            def _():
                c0 = pl.multiple_of(i * _PC, 128)
                accc_ref[...] += cached_partial(c0, _PC)

            @pl.when(i == NCH - 1)
            def _():
                accc_ref[...] += cached_partial(_PC * (NCH - 1), WLAST)

            @pl.when(i < R - _RC)
            def _():
                finish((i + _RC) * _BI, jax.lax.dot_general(
                    adj_ref[...], g16_ref[...], dc,
                    preferred_element_type=jnp.float32))

            @pl.when(i >= R - _RC)
            def _():
                blk = i - (R - _RC)
                finish(blk * _BI,
                       accc_ref[pl.ds(blk * _BI, _BI), :])

        return None

    return body


def kernel(features, adj, W1, b1, W2, b2):
    N, H = features.shape
    R = N // _BI
    bias = (b1 + b2).reshape(1, H)

    def adj_map(p, i):
        return (jnp.where(p == 0, i, jnp.minimum(i + _RC, R - 1)), 0)

    def out_map(p, i):
        return (p * jnp.where(i < R - _RC, i + _RC, i - (R - _RC)), 0)

    pre, out = pl.pallas_call(
        _make_body(N, R),
        grid=(2, R),
        in_specs=[
            pl.BlockSpec((_BI, N), adj_map),
            pl.BlockSpec((N, H), lambda p, i: (0, 0)),
            pl.BlockSpec((H, H), lambda p, i: (0, 0)),
            pl.BlockSpec((H, H), lambda p, i: (0, 0)),
            pl.BlockSpec((1, H), lambda p, i: (0, 0)),
        ],
        out_specs=[
            pl.BlockSpec((_BI, H), out_map),
            pl.BlockSpec((_BI, H), out_map),
        ],
        out_shape=[
            jax.ShapeDtypeStruct((N, H), jnp.float32),
            jax.ShapeDtypeStruct((N, H), jnp.float32),
        ],
        scratch_shapes=[
            pltpu.VMEM((_RC * _BI, N), jnp.bfloat16),
            pltpu.VMEM((N, H), jnp.bfloat16),
            pltpu.VMEM((N, H), jnp.bfloat16),
            pltpu.VMEM((_RC * _BI, H), jnp.float32),
        ],
    )(adj, features, W1, W2, bias)
    return (pre, out)
